# Initial kernel scaffold; baseline (speedup 1.0000x reference)
import jax
import jax.numpy as jnp
from jax.experimental import pallas as pl


def kernel(x, Wg, W1, b1, ln_g, ln_b, W2, b2):
    raise NotImplementedError("write your pallas kernel here")


# speedup vs baseline: 1.2222x; 1.2222x over previous
import functools

import jax
import jax.numpy as jnp
from jax.experimental import pallas as pl
from jax.experimental.pallas import tpu as pltpu

DIM = 768
E = 8
K = 2
DFF = DIM * 4
EPS = 1e-5
N_TOK = 2048
BT = 256


def _moe_body(x32_ref, xb_ref, Wg_ref, W1_ref, b1_ref, g_ref, bln_ref,
              W2_ref, b2_ref, out_ref, wfull_ref):
    e = pl.program_id(0)
    t = pl.program_id(1)
    rows = pl.ds(t * BT, BT)

    @pl.when(e == 0)
    def _gating():
        logits = jnp.dot(x32_ref[rows, :], Wg_ref[...],
                         preferred_element_type=jnp.float32)
        eidx = jax.lax.broadcasted_iota(jnp.int32, (BT, E), 1)
        m1 = jnp.max(logits, axis=1, keepdims=True)
        i1 = jnp.argmax(logits, axis=1).reshape(BT, 1)
        masked = jnp.where(eidx == i1, -jnp.inf, logits)
        m2 = jnp.max(masked, axis=1, keepdims=True)
        i2 = jnp.argmax(masked, axis=1).reshape(BT, 1)
        z = jnp.exp(m2 - m1)
        w1 = 1.0 / (1.0 + z)
        w2 = z / (1.0 + z)
        wfull_ref[rows, :] = jnp.where(eidx == i1, w1,
                                       jnp.where(eidx == i2, w2, 0.0))

    eidx = jax.lax.broadcasted_iota(jnp.int32, (BT, E), 1)
    wi = jnp.sum(wfull_ref[rows, :] * (eidx == e).astype(jnp.float32),
                 axis=1, keepdims=True)

    h = jnp.dot(xb_ref[rows, :], W1_ref[0],
                preferred_element_type=jnp.float32) + b1_ref[0, 0]
    h = h * jax.nn.sigmoid(h)
    mu = jnp.mean(h, axis=1, keepdims=True)
    hc = h - mu
    var = jnp.mean(hc * hc, axis=1, keepdims=True)
    h = hc * jax.lax.rsqrt(var + EPS) * g_ref[0, 0] + bln_ref[0, 0]
    y = jnp.dot(h.astype(jnp.bfloat16), W2_ref[0],
                preferred_element_type=jnp.float32) + b2_ref[0, 0]
    contrib = wi * y

    @pl.when(e == 0)
    def _init():
        out_ref[rows, :] = contrib

    @pl.when(e > 0)
    def _acc():
        out_ref[rows, :] += contrib


@jax.jit
def kernel(x, Wg, W1, b1, ln_g, ln_b, W2, b2):
    xb = x.astype(jnp.bfloat16)
    W1b = W1.astype(jnp.bfloat16)
    W2b = W2.astype(jnp.bfloat16)
    T = N_TOK // BT
    grid = (E, T)
    out = pl.pallas_call(
        _moe_body,
        grid=grid,
        in_specs=[
            pl.BlockSpec((N_TOK, DIM), lambda e, t: (0, 0)),
            pl.BlockSpec((N_TOK, DIM), lambda e, t: (0, 0)),
            pl.BlockSpec((DIM, E), lambda e, t: (0, 0)),
            pl.BlockSpec((1, DIM, DFF), lambda e, t: (e, 0, 0)),
            pl.BlockSpec((1, 1, DFF), lambda e, t: (e, 0, 0)),
            pl.BlockSpec((1, 1, DFF), lambda e, t: (e, 0, 0)),
            pl.BlockSpec((1, 1, DFF), lambda e, t: (e, 0, 0)),
            pl.BlockSpec((1, DFF, DIM), lambda e, t: (e, 0, 0)),
            pl.BlockSpec((1, 1, DIM), lambda e, t: (e, 0, 0)),
        ],
        out_specs=pl.BlockSpec((N_TOK, DIM), lambda e, t: (0, 0)),
        out_shape=jax.ShapeDtypeStruct((N_TOK, DIM), jnp.float32),
        scratch_shapes=[pltpu.VMEM((N_TOK, E), jnp.float32)],
        compiler_params=pltpu.CompilerParams(
            dimension_semantics=("arbitrary", "arbitrary"),
        ),
    )(x, xb, Wg, W1b,
      b1.reshape(E, 1, DFF), ln_g.reshape(E, 1, DFF),
      ln_b.reshape(E, 1, DFF), W2b, b2.reshape(E, 1, DIM))
    return out


# speedup vs baseline: 1.4445x; 1.1819x over previous
import functools

import jax
import jax.numpy as jnp
from jax.experimental import pallas as pl
from jax.experimental.pallas import tpu as pltpu

DIM = 768
E = 8
K = 2
DFF = DIM * 4
EPS = 1e-5
N_TOK = 2048
BT = 256


def _moe_body(x32_ref, Wg_ref, W1_ref, b1_ref, g_ref, bln_ref,
              W2_ref, b2_ref, out_ref, wfull_ref):
    e = pl.program_id(0)
    t = pl.program_id(1)
    rows = pl.ds(t * BT, BT)

    @pl.when(e == 0)
    def _gating():
        logits = jnp.dot(x32_ref[rows, :], Wg_ref[...],
                         preferred_element_type=jnp.float32)
        eidx = jax.lax.broadcasted_iota(jnp.int32, (BT, E), 1)
        m1 = jnp.max(logits, axis=1, keepdims=True)
        i1 = jnp.argmax(logits, axis=1).reshape(BT, 1)
        masked = jnp.where(eidx == i1, -jnp.inf, logits)
        m2 = jnp.max(masked, axis=1, keepdims=True)
        i2 = jnp.argmax(masked, axis=1).reshape(BT, 1)
        z = jnp.exp(m2 - m1)
        w1 = 1.0 / (1.0 + z)
        w2 = z / (1.0 + z)
        wfull_ref[rows, :] = jnp.where(eidx == i1, w1,
                                       jnp.where(eidx == i2, w2, 0.0))

    eidx = jax.lax.broadcasted_iota(jnp.int32, (BT, E), 1)
    wi = jnp.sum(wfull_ref[rows, :] * (eidx == e).astype(jnp.float32),
                 axis=1, keepdims=True)

    h = jnp.dot(x32_ref[rows, :], W1_ref[0],
                preferred_element_type=jnp.float32) + b1_ref[0, 0]
    h = h * jax.nn.sigmoid(h)
    mu = jnp.mean(h, axis=1, keepdims=True)
    hc = h - mu
    var = jnp.mean(hc * hc, axis=1, keepdims=True)
    h = hc * jax.lax.rsqrt(var + EPS) * g_ref[0, 0] + bln_ref[0, 0]
    y = jnp.dot(h, W2_ref[0],
                preferred_element_type=jnp.float32) + b2_ref[0, 0]
    contrib = wi * y

    @pl.when(e == 0)
    def _init():
        out_ref[rows, :] = contrib

    @pl.when(e > 0)
    def _acc():
        out_ref[rows, :] += contrib


@jax.jit
def kernel(x, Wg, W1, b1, ln_g, ln_b, W2, b2):
    T = N_TOK // BT
    grid = (E, T)
    out = pl.pallas_call(
        _moe_body,
        grid=grid,
        in_specs=[
            pl.BlockSpec((N_TOK, DIM), lambda e, t: (0, 0)),
            pl.BlockSpec((DIM, E), lambda e, t: (0, 0)),
            pl.BlockSpec((1, DIM, DFF), lambda e, t: (e, 0, 0)),
            pl.BlockSpec((1, 1, DFF), lambda e, t: (e, 0, 0)),
            pl.BlockSpec((1, 1, DFF), lambda e, t: (e, 0, 0)),
            pl.BlockSpec((1, 1, DFF), lambda e, t: (e, 0, 0)),
            pl.BlockSpec((1, DFF, DIM), lambda e, t: (e, 0, 0)),
            pl.BlockSpec((1, 1, DIM), lambda e, t: (e, 0, 0)),
        ],
        out_specs=pl.BlockSpec((N_TOK, DIM), lambda e, t: (0, 0)),
        out_shape=jax.ShapeDtypeStruct((N_TOK, DIM), jnp.float32),
        scratch_shapes=[pltpu.VMEM((N_TOK, E), jnp.float32)],
        compiler_params=pltpu.CompilerParams(
            dimension_semantics=("arbitrary", "arbitrary"),
        ),
    )(x, Wg, W1,
      b1.reshape(E, 1, DFF), ln_g.reshape(E, 1, DFF),
      ln_b.reshape(E, 1, DFF), W2, b2.reshape(E, 1, DIM))
    return out


# speedup vs baseline: 2.0687x; 1.4321x over previous
import functools

import jax
import jax.numpy as jnp
from jax.experimental import pallas as pl
from jax.experimental.pallas import tpu as pltpu

DIM = 768
E = 8
K = 2
DFF = DIM * 4
EPS = 1e-5
N_TOK = 2048
BT = 256
GMAX = 32


def _routing_body(x_ref, Wg_ref, wT_ref, R_ref, meta_ref):
    logits = jnp.dot(x_ref[...], Wg_ref[...],
                     preferred_element_type=jnp.float32)
    eidx = jax.lax.broadcasted_iota(jnp.int32, (N_TOK, E), 1)
    m1 = jnp.max(logits, axis=1, keepdims=True)
    i1 = jnp.argmax(logits, axis=1).reshape(N_TOK, 1)
    masked = jnp.where(eidx == i1, -jnp.inf, logits)
    i2 = jnp.argmax(masked, axis=1).reshape(N_TOK, 1)
    m2 = jnp.max(masked, axis=1, keepdims=True)
    z = jnp.exp(m2 - m1)
    w1 = 1.0 / (1.0 + z)
    w2 = z / (1.0 + z)
    wfull = jnp.where(eidx == i1, w1, jnp.where(eidx == i2, w2, 0.0))
    maskf = jnp.where(eidx == i1, 1.0, jnp.where(eidx == i2, 1.0, 0.0))

    r_io = jax.lax.broadcasted_iota(jnp.int32, (N_TOK, N_TOK), 0)
    c_io = jax.lax.broadcasted_iota(jnp.int32, (N_TOK, N_TOK), 1)
    Lstrict = (c_io < r_io).astype(jnp.bfloat16)
    rank = jnp.dot(Lstrict, maskf.astype(jnp.bfloat16),
                   preferred_element_type=jnp.float32)
    rankT = rank.T.astype(jnp.int32)
    wT = wfull.T
    maskT = maskf.T.astype(jnp.int32)
    R = jnp.where(maskT == 1, rankT, -1)
    counts = jnp.sum(maskT, axis=1, keepdims=True)

    num_tiles = (counts + (BT - 1)) // BT
    tri_r = jax.lax.broadcasted_iota(jnp.int32, (E, E), 0)
    tri_c = jax.lax.broadcasted_iota(jnp.int32, (E, E), 1)
    tri8 = (tri_c <= tri_r).astype(jnp.float32)
    cumT = jnp.dot(tri8, num_tiles.astype(jnp.float32),
                   preferred_element_type=jnp.float32).astype(jnp.int32)
    cumT_ex = cumT - num_tiles
    g_iota = jax.lax.broadcasted_iota(jnp.int32, (E, GMAX), 1)
    te = jnp.sum((g_iota >= cumT).astype(jnp.int32), axis=0, keepdims=True)
    te = jnp.minimum(te, E - 1)
    e_iota = jax.lax.broadcasted_iota(jnp.int32, (E, GMAX), 0)
    cumT_ex_g = jnp.sum(jnp.where(e_iota == te, cumT_ex, 0),
                        axis=0, keepdims=True)
    g_row = jax.lax.broadcasted_iota(jnp.int32, (1, GMAX), 1)
    p0 = (g_row - cumT_ex_g) * BT
    real = (g_row < cumT[E - 1:E, :]).astype(jnp.int32)

    wT_ref[...] = wT.astype(jnp.bfloat16).reshape(E, 1, N_TOK)
    R_ref[...] = R.reshape(E, 1, N_TOK)
    meta_ref[0:1, :] = te
    meta_ref[1:2, :] = p0
    meta_ref[2:3, :] = real


def _ffn_body(sp_ref, xb_ref, wT_ref, R_ref, W1_ref, b1_ref, g_ref, bln_ref,
              W2_ref, b2_ref, out_ref):
    g = pl.program_id(0)
    p0 = sp_ref[GMAX + g]
    real = sp_ref[2 * GMAX + g]

    @pl.when(g == 0)
    def _init():
        out_ref[...] = jnp.zeros_like(out_ref)

    @pl.when(real == 1)
    def _compute():
        Rb = jnp.broadcast_to(R_ref[0], (BT, N_TOK))
        target = jax.lax.broadcasted_iota(jnp.int32, (BT, N_TOK), 0) + p0
        G = (Rb == target).astype(jnp.bfloat16)
        Gw = G * jnp.broadcast_to(wT_ref[0], (BT, N_TOK))
        xg = jnp.dot(G, xb_ref[...],
                     preferred_element_type=jnp.float32)
        xg = xg.astype(jnp.bfloat16)
        h = jnp.dot(xg, W1_ref[0],
                    preferred_element_type=jnp.float32) + b1_ref[0, 0]
        h = h * jax.nn.sigmoid(h)
        mu = jnp.mean(h, axis=1, keepdims=True)
        hc = h - mu
        var = jnp.mean(hc * hc, axis=1, keepdims=True)
        h = hc * jax.lax.rsqrt(var + EPS) * g_ref[0, 0] + bln_ref[0, 0]
        y = jnp.dot(h.astype(jnp.bfloat16), W2_ref[0],
                    preferred_element_type=jnp.float32) + b2_ref[0, 0]
        contrib = jax.lax.dot_general(
            Gw, y.astype(jnp.bfloat16),
            dimension_numbers=(((0,), (0,)), ((), ())),
            preferred_element_type=jnp.float32)
        out_ref[...] += contrib


@jax.jit
def kernel(x, Wg, W1, b1, ln_g, ln_b, W2, b2):
    wTb, R, meta = pl.pallas_call(
        _routing_body,
        out_shape=[
            jax.ShapeDtypeStruct((E, 1, N_TOK), jnp.bfloat16),
            jax.ShapeDtypeStruct((E, 1, N_TOK), jnp.int32),
            jax.ShapeDtypeStruct((3, GMAX), jnp.int32),
        ],
    )(x, Wg)
    sp = meta.reshape(3 * GMAX)
    xb = x.astype(jnp.bfloat16)
    W1b = W1.astype(jnp.bfloat16)
    W2b = W2.astype(jnp.bfloat16)

    grid_spec = pltpu.PrefetchScalarGridSpec(
        num_scalar_prefetch=1,
        grid=(GMAX,),
        in_specs=[
            pl.BlockSpec((N_TOK, DIM), lambda g, sp: (0, 0)),
            pl.BlockSpec((1, 1, N_TOK), lambda g, sp: (sp[g], 0, 0)),
            pl.BlockSpec((1, 1, N_TOK), lambda g, sp: (sp[g], 0, 0)),
            pl.BlockSpec((1, DIM, DFF), lambda g, sp: (sp[g], 0, 0)),
            pl.BlockSpec((1, 1, DFF), lambda g, sp: (sp[g], 0, 0)),
            pl.BlockSpec((1, 1, DFF), lambda g, sp: (sp[g], 0, 0)),
            pl.BlockSpec((1, 1, DFF), lambda g, sp: (sp[g], 0, 0)),
            pl.BlockSpec((1, DFF, DIM), lambda g, sp: (sp[g], 0, 0)),
            pl.BlockSpec((1, 1, DIM), lambda g, sp: (sp[g], 0, 0)),
        ],
        out_specs=pl.BlockSpec((N_TOK, DIM), lambda g, sp: (0, 0)),
    )
    out = pl.pallas_call(
        _ffn_body,
        grid_spec=grid_spec,
        out_shape=jax.ShapeDtypeStruct((N_TOK, DIM), jnp.float32),
        compiler_params=pltpu.CompilerParams(
            dimension_semantics=("arbitrary",),
        ),
    )(sp, xb, wTb, R, W1b,
      b1.reshape(E, 1, DFF), ln_g.reshape(E, 1, DFF),
      ln_b.reshape(E, 1, DFF), W2b, b2.reshape(E, 1, DIM))
    return out


# speedup vs baseline: 2.9926x; 1.4467x over previous
import functools

import jax
import jax.numpy as jnp
from jax.experimental import pallas as pl
from jax.experimental.pallas import tpu as pltpu

DIM = 768
E = 8
K = 2
DFF = DIM * 4
EPS = 1e-5
N_TOK = 2048
BT = 256
GMAX = 24


def _routing_body(x_ref, Wg_ref, wT_ref, R_ref, meta_ref):
    logits = jnp.dot(x_ref[...], Wg_ref[...],
                     preferred_element_type=jnp.float32)
    eidx = jax.lax.broadcasted_iota(jnp.int32, (N_TOK, E), 1)
    m1 = jnp.max(logits, axis=1, keepdims=True)
    i1 = jnp.argmax(logits, axis=1).reshape(N_TOK, 1)
    masked = jnp.where(eidx == i1, -jnp.inf, logits)
    i2 = jnp.argmax(masked, axis=1).reshape(N_TOK, 1)
    m2 = jnp.max(masked, axis=1, keepdims=True)
    z = jnp.exp(m2 - m1)
    w1 = 1.0 / (1.0 + z)
    w2 = z / (1.0 + z)
    wfull = jnp.where(eidx == i1, w1, jnp.where(eidx == i2, w2, 0.0))
    maskf = jnp.where(eidx == i1, 1.0, jnp.where(eidx == i2, 1.0, 0.0))

    r_io = jax.lax.broadcasted_iota(jnp.int32, (N_TOK, N_TOK), 0)
    c_io = jax.lax.broadcasted_iota(jnp.int32, (N_TOK, N_TOK), 1)
    Lstrict = (c_io < r_io).astype(jnp.bfloat16)
    rank = jnp.dot(Lstrict, maskf.astype(jnp.bfloat16),
                   preferred_element_type=jnp.float32)
    rankT = rank.T.astype(jnp.int32)
    wT = wfull.T
    maskT = maskf.T.astype(jnp.int32)
    R = jnp.where(maskT == 1, rankT, -1)
    counts = jnp.sum(maskT, axis=1, keepdims=True)

    num_tiles = (counts + (BT - 1)) // BT
    tri_r = jax.lax.broadcasted_iota(jnp.int32, (E, E), 0)
    tri_c = jax.lax.broadcasted_iota(jnp.int32, (E, E), 1)
    tri8 = (tri_c <= tri_r).astype(jnp.float32)
    cumT = jnp.dot(tri8, num_tiles.astype(jnp.float32),
                   preferred_element_type=jnp.float32).astype(jnp.int32)
    cumT_ex = cumT - num_tiles
    g_iota = jax.lax.broadcasted_iota(jnp.int32, (E, GMAX), 1)
    te = jnp.sum((g_iota >= cumT).astype(jnp.int32), axis=0, keepdims=True)
    te = jnp.minimum(te, E - 1)
    e_iota = jax.lax.broadcasted_iota(jnp.int32, (E, GMAX), 0)
    cumT_ex_g = jnp.sum(jnp.where(e_iota == te, cumT_ex, 0),
                        axis=0, keepdims=True)
    g_row = jax.lax.broadcasted_iota(jnp.int32, (1, GMAX), 1)
    p0 = (g_row - cumT_ex_g) * BT
    real = (g_row < cumT[E - 1:E, :]).astype(jnp.int32)

    wT_ref[...] = wT.reshape(E, 1, N_TOK)
    R_ref[...] = R.reshape(E, 1, N_TOK)
    meta_ref[0:1, :] = te
    meta_ref[1:2, :] = p0
    meta_ref[2:3, :] = real


def _ffn_body(sp_ref, x_ref, wT_ref, R_ref, W1_ref, b1_ref, g_ref, bln_ref,
              W2_ref, b2_ref, out_ref):
    g = pl.program_id(0)
    p0 = sp_ref[GMAX + g]
    real = sp_ref[2 * GMAX + g]

    @pl.when(g == 0)
    def _init():
        out_ref[...] = jnp.zeros_like(out_ref)

    @pl.when(real == 1)
    def _compute():
        Rb = jnp.broadcast_to(R_ref[0], (BT, N_TOK))
        target = jax.lax.broadcasted_iota(jnp.int32, (BT, N_TOK), 0) + p0
        G = (Rb == target).astype(jnp.bfloat16)
        Gw = G * jnp.broadcast_to(wT_ref[0], (BT, N_TOK)).astype(jnp.bfloat16)
        xg = jnp.dot(G.astype(jnp.float32), x_ref[...],
                     preferred_element_type=jnp.float32)
        h = jnp.dot(xg, W1_ref[0],
                    preferred_element_type=jnp.float32) + b1_ref[0, 0]
        h = h * jax.nn.sigmoid(h)
        mu = jnp.mean(h, axis=1, keepdims=True)
        hc = h - mu
        var = jnp.mean(hc * hc, axis=1, keepdims=True)
        h = hc * jax.lax.rsqrt(var + EPS) * g_ref[0, 0] + bln_ref[0, 0]
        y = jnp.dot(h, W2_ref[0],
                    preferred_element_type=jnp.float32) + b2_ref[0, 0]
        contrib = jax.lax.dot_general(
            Gw, y.astype(jnp.bfloat16),
            dimension_numbers=(((0,), (0,)), ((), ())),
            preferred_element_type=jnp.float32)
        out_ref[...] += contrib


@jax.jit
def kernel(x, Wg, W1, b1, ln_g, ln_b, W2, b2):
    wT, R, meta = pl.pallas_call(
        _routing_body,
        out_shape=[
            jax.ShapeDtypeStruct((E, 1, N_TOK), jnp.float32),
            jax.ShapeDtypeStruct((E, 1, N_TOK), jnp.int32),
            jax.ShapeDtypeStruct((3, GMAX), jnp.int32),
        ],
    )(x, Wg)
    sp = meta.reshape(3 * GMAX)

    grid_spec = pltpu.PrefetchScalarGridSpec(
        num_scalar_prefetch=1,
        grid=(GMAX,),
        in_specs=[
            pl.BlockSpec((N_TOK, DIM), lambda g, sp: (0, 0)),
            pl.BlockSpec((1, 1, N_TOK), lambda g, sp: (sp[g], 0, 0)),
            pl.BlockSpec((1, 1, N_TOK), lambda g, sp: (sp[g], 0, 0)),
            pl.BlockSpec((1, DIM, DFF), lambda g, sp: (sp[g], 0, 0)),
            pl.BlockSpec((1, 1, DFF), lambda g, sp: (sp[g], 0, 0)),
            pl.BlockSpec((1, 1, DFF), lambda g, sp: (sp[g], 0, 0)),
            pl.BlockSpec((1, 1, DFF), lambda g, sp: (sp[g], 0, 0)),
            pl.BlockSpec((1, DFF, DIM), lambda g, sp: (sp[g], 0, 0)),
            pl.BlockSpec((1, 1, DIM), lambda g, sp: (sp[g], 0, 0)),
        ],
        out_specs=pl.BlockSpec((N_TOK, DIM), lambda g, sp: (0, 0)),
    )
    out = pl.pallas_call(
        _ffn_body,
        grid_spec=grid_spec,
        out_shape=jax.ShapeDtypeStruct((N_TOK, DIM), jnp.float32),
        compiler_params=pltpu.CompilerParams(
            dimension_semantics=("arbitrary",),
        ),
    )(sp, x, wT, R, W1,
      b1.reshape(E, 1, DFF), ln_g.reshape(E, 1, DFF),
      ln_b.reshape(E, 1, DFF), W2, b2.reshape(E, 1, DIM))
    return out
